# Initial kernel scaffold; baseline (speedup 1.0000x reference)
import jax
import jax.numpy as jnp
from jax.experimental import pallas as pl


def kernel(xyz, feats, W1, b1, g1, be1, W2, b2, Wv, bv, gv, bev, Wo, bo, go, beo):
    raise NotImplementedError("write your pallas kernel here")


# speedup vs baseline: 7.1440x; 7.1440x over previous
import functools

import jax
import jax.numpy as jnp
from jax import lax
from jax.experimental import pallas as pl
from jax.experimental.pallas import tpu as pltpu
from jax.experimental.pallas import tpu_sc as plsc

_B, _N, _C, _H, _K = 8, 1024, 256, 64, 16
_BN = _B * _N
_KBN = _K * _BN
_F32 = jnp.float32
_NEG = -3.0e38

_HI = lax.Precision.HIGHEST


def _proj_body(ft, xt, w1a, w1b, w1c, wva, wvb, b1, bv, a_o, cc_o, vn_o, vc_o):
    f = ft[...]
    x = xt[...]
    xc1 = lax.dot(x, w1c[...], precision=_HI)
    xcv = lax.dot(x, wvb[...], precision=_HI)
    a_o[...] = lax.dot(f, w1b[...], precision=_HI) + xc1
    cc_o[...] = lax.dot(f, w1a[...], precision=_HI) - xc1 + b1[...]
    vn_o[...] = lax.dot(f, wva[...], precision=_HI) + xcv
    vc_o[...] = bv[...] - xcv


def _projections(ft, xt, w1a, w1b, w1c, wva, wvb, b1, bv):
    rp = 512
    grid = (_BN // rp,)
    full = lambda a: pl.BlockSpec(a.shape, lambda t: (0,) * a.ndim)
    row = lambda d: pl.BlockSpec((rp, d), lambda t: (t, 0))
    return pl.pallas_call(
        _proj_body,
        grid=grid,
        in_specs=[row(_C), row(3), full(w1a), full(w1b), full(w1c),
                  full(wva), full(wvb), full(b1), full(bv)],
        out_specs=[row(_H), row(_H), row(_C), row(_C)],
        out_shape=[
            jax.ShapeDtypeStruct((_BN, _H), _F32),
            jax.ShapeDtypeStruct((_BN, _H), _F32),
            jax.ShapeDtypeStruct((_BN, _C), _F32),
            jax.ShapeDtypeStruct((_BN, _C), _F32),
        ],
    )(ft, xt, w1a, w1b, w1c, wva, wvb, b1, bv)


def _knn_body(xb, xt, idx_o):
    b = pl.program_id(0)
    x_b = xb[0]
    x_t = xt[...]
    g = lax.dot(x_t, x_b, precision=_HI)
    sqc = jnp.sum(x_b * x_b, axis=0, keepdims=True)
    sqr = jnp.sum(x_t * x_t, axis=1, keepdims=True)
    s = -((-2.0 * g + sqr) + sqc)
    iota = lax.broadcasted_iota(jnp.int32, s.shape, 1)
    cols = []
    for _ in range(_K):
        m = jnp.max(s, axis=1, keepdims=True)
        j = jnp.min(jnp.where(s == m, iota, _N), axis=1, keepdims=True)
        cols.append(j)
        s = jnp.where(iota == j, _NEG, s)
    idx_o[...] = jnp.concatenate(cols, axis=1) + b * _N


def _knn(xyz, xt):
    rt = 256
    tpb = _N // rt
    return pl.pallas_call(
        _knn_body,
        grid=(_B, tpb),
        in_specs=[
            pl.BlockSpec((1, 3, _N), lambda b, t: (b, 0, 0)),
            pl.BlockSpec((rt, 3), lambda b, t: (b * tpb + t, 0)),
        ],
        out_specs=pl.BlockSpec((rt, _K), lambda b, t: (b * tpb + t, 0)),
        out_shape=jax.ShapeDtypeStruct((_BN, _K), jnp.int32),
    )(xyz, xt)


def _sc_gather(idxf, a_tab, v_tab):
    info = plsc.get_sparse_core_info()
    nc, ns = info.num_cores, info.num_subcores
    nw = nc * ns
    ch = 128
    per_w = _KBN // nw
    nch = per_w // ch
    mesh = plsc.VectorSubcoreMesh(core_axis_name="c", subcore_axis_name="s")

    @functools.partial(
        pl.kernel,
        mesh=mesh,
        compiler_params=pltpu.CompilerParams(use_tc_tiling_on_sc=False),
        out_type=(
            jax.ShapeDtypeStruct((_KBN, _H), _F32),
            jax.ShapeDtypeStruct((_KBN, _C), _F32),
        ),
        scratch_types=[
            pltpu.VMEM((ch,), jnp.int32),
            pltpu.VMEM((ch, _H), _F32),
            pltpu.VMEM((ch, _C), _F32),
            pltpu.SemaphoreType.DMA,
            pltpu.SemaphoreType.DMA,
        ],
    )
    def gathk(idx_hbm, a_hbm, v_hbm, ag_hbm, vg_hbm, idx_v, a_v, v_v, s1, s2):
        wid = lax.axis_index("s") * nc + lax.axis_index("c")

        @pl.loop(0, nch)
        def _chunk(c):
            base = wid * per_w + c * ch
            pltpu.sync_copy(idx_hbm.at[pl.ds(base, ch)], idx_v)
            ca = pltpu.async_copy(a_hbm.at[idx_v], a_v, s1)
            cv = pltpu.async_copy(v_hbm.at[idx_v], v_v, s2)
            ca.wait()
            cv.wait()
            pltpu.sync_copy(a_v, ag_hbm.at[pl.ds(base, ch)])
            pltpu.sync_copy(v_v, vg_hbm.at[pl.ds(base, ch)])

    return gathk(idxf, a_tab, v_tab)


def _stats_body(ag, vg, cc, vc, o1, ov):
    @pl.when(pl.program_id(0) == 0)
    def _():
        o1[...] = jnp.zeros_like(o1)
        ov[...] = jnp.zeros_like(ov)

    c = cc[...]
    v = vc[...]
    su1 = jnp.zeros((1, _H), _F32)
    sq1 = jnp.zeros((1, _H), _F32)
    suv = jnp.zeros((1, _C), _F32)
    sqv = jnp.zeros((1, _C), _F32)
    for k in range(_K):
        x1 = ag[k] + c
        su1 += jnp.sum(x1, axis=0, keepdims=True)
        sq1 += jnp.sum(x1 * x1, axis=0, keepdims=True)
        v1 = vg[k] + v
        suv += jnp.sum(v1, axis=0, keepdims=True)
        sqv += jnp.sum(v1 * v1, axis=0, keepdims=True)
    z1 = jnp.zeros((6, _H), _F32)
    zv = jnp.zeros((6, _C), _F32)
    o1[...] += jnp.concatenate([su1, sq1, z1], axis=0)
    ov[...] += jnp.concatenate([suv, sqv, zv], axis=0)


def _stats(ag, vg, cc, vc):
    rs = 512
    grid = (_BN // rs,)
    return pl.pallas_call(
        _stats_body,
        grid=grid,
        in_specs=[
            pl.BlockSpec((_K, rs, _H), lambda t: (0, t, 0)),
            pl.BlockSpec((_K, rs, _C), lambda t: (0, t, 0)),
            pl.BlockSpec((rs, _H), lambda t: (t, 0)),
            pl.BlockSpec((rs, _C), lambda t: (t, 0)),
        ],
        out_specs=[
            pl.BlockSpec((8, _H), lambda t: (0, 0)),
            pl.BlockSpec((8, _C), lambda t: (0, 0)),
        ],
        out_shape=[
            jax.ShapeDtypeStruct((8, _H), _F32),
            jax.ShapeDtypeStruct((8, _C), _F32),
        ],
    )(ag, vg, cc, vc)


def _main_body(ag, vg, cc, vc, sc1, sh1, scv, shv, w2, wo, bo, z_o, zs_o):
    c = cc[...]
    v = vc[...]
    a1 = sc1[...]
    b1 = sh1[...]
    av = scv[...]
    bv = shv[...]
    w2r = w2[...]
    logits = []
    for k in range(_K):
        h = jnp.maximum((ag[k] + c) * a1 + b1, 0.0)
        logits.append(jnp.sum(h * w2r, axis=1, keepdims=True))
    lg = jnp.concatenate(logits, axis=1)
    mx = jnp.max(lg, axis=1, keepdims=True)
    e = jnp.exp(lg - mx)
    attn = e / jnp.sum(e, axis=1, keepdims=True)
    out = jnp.zeros(v.shape, _F32)
    for k in range(_K):
        vals = jnp.maximum((vg[k] + v) * av + bv, 0.0)
        out += attn[:, k:k + 1] * vals
    z = lax.dot(out, wo[...], precision=_HI) + bo[...]
    z_o[...] = z

    @pl.when(pl.program_id(0) == 0)
    def _():
        zs_o[...] = jnp.zeros_like(zs_o)

    zs_o[...] += jnp.concatenate(
        [jnp.sum(z, axis=0, keepdims=True),
         jnp.sum(z * z, axis=0, keepdims=True),
         jnp.zeros((6, _C), _F32)], axis=0)


def _main(ag, vg, cc, vc, sc1, sh1, scv, shv, w2, wo, bo):
    rm = 256
    grid = (_BN // rm,)
    full = lambda a: pl.BlockSpec(a.shape, lambda t: (0,) * a.ndim)
    return pl.pallas_call(
        _main_body,
        grid=grid,
        in_specs=[
            pl.BlockSpec((_K, rm, _H), lambda t: (0, t, 0)),
            pl.BlockSpec((_K, rm, _C), lambda t: (0, t, 0)),
            pl.BlockSpec((rm, _H), lambda t: (t, 0)),
            pl.BlockSpec((rm, _C), lambda t: (t, 0)),
            full(sc1), full(sh1), full(scv), full(shv), full(w2),
            full(wo), full(bo),
        ],
        out_specs=[
            pl.BlockSpec((rm, _C), lambda t: (t, 0)),
            pl.BlockSpec((8, _C), lambda t: (0, 0)),
        ],
        out_shape=[
            jax.ShapeDtypeStruct((_BN, _C), _F32),
            jax.ShapeDtypeStruct((8, _C), _F32),
        ],
    )(ag, vg, cc, vc, sc1, sh1, scv, shv, w2, wo, bo)


def _epi_body(z, ft, sco, sho, o_o):
    o_o[...] = jnp.maximum(z[...] * sco[...] + sho[...], 0.0) + ft[...]


def _epilogue(z, ft, sco, sho):
    re = 512
    full = lambda a: pl.BlockSpec(a.shape, lambda t: (0,) * a.ndim)
    return pl.pallas_call(
        _epi_body,
        grid=(_BN // re,),
        in_specs=[
            pl.BlockSpec((re, _C), lambda t: (t, 0)),
            pl.BlockSpec((re, _C), lambda t: (t, 0)),
            full(sco), full(sho),
        ],
        out_specs=pl.BlockSpec((re, _C), lambda t: (t, 0)),
        out_shape=jax.ShapeDtypeStruct((_BN, _C), _F32),
    )(z, ft, sco, sho)


def _bn_coef(s, q, m, g, be):
    mean = s / m
    var = q / m - mean * mean
    sc = g / jnp.sqrt(var + 1e-5)
    return sc, be - mean * sc


def kernel(xyz, feats, W1, b1, g1, be1, W2, b2, Wv, bv, gv, bev, Wo, bo, go, beo):
    ft = jnp.transpose(feats, (0, 2, 1)).reshape(_BN, _C)
    xt = jnp.transpose(xyz, (0, 2, 1)).reshape(_BN, 3)

    w1a = jnp.transpose(W1[:, :_C])
    w1b = jnp.transpose(W1[:, _C:2 * _C])
    w1c = jnp.transpose(W1[:, 2 * _C:])
    wva = jnp.transpose(Wv[:, :_C])
    wvb = jnp.transpose(Wv[:, _C:])

    a_tab, cc_tab, vn_tab, vc_tab = _projections(
        ft, xt, w1a, w1b, w1c, wva, wvb,
        b1.reshape(1, _H), bv.reshape(1, _C))

    idx = _knn(xyz, xt)
    idxf = jnp.transpose(idx).reshape(_KBN)

    ag, vg = _sc_gather(idxf, a_tab, vn_tab)
    ag = ag.reshape(_K, _BN, _H)
    vg = vg.reshape(_K, _BN, _C)

    o1, ov = _stats(ag, vg, cc_tab, vc_tab)
    m1 = float(_KBN)
    sc1, sh1 = _bn_coef(o1[0:1], o1[1:2], m1, g1.reshape(1, _H), be1.reshape(1, _H))
    scv, shv = _bn_coef(ov[0:1], ov[1:2], m1, gv.reshape(1, _C), bev.reshape(1, _C))

    z, zs = _main(ag, vg, cc_tab, vc_tab, sc1, sh1, scv, shv,
                  W2.reshape(1, _H), jnp.transpose(Wo), bo.reshape(1, _C))
    sco, sho = _bn_coef(zs[0:1], zs[1:2], float(_BN),
                        go.reshape(1, _C), beo.reshape(1, _C))

    o = _epilogue(z, ft, sco, sho)
    return jnp.transpose(o.reshape(_B, _N, _C), (0, 2, 1))


# speedup vs baseline: 10.6966x; 1.4973x over previous
import functools

import jax
import jax.numpy as jnp
from jax import lax
from jax.experimental import pallas as pl
from jax.experimental.pallas import tpu as pltpu
from jax.experimental.pallas import tpu_sc as plsc

_B, _N, _C, _H, _K = 8, 1024, 256, 64, 16
_BN = _B * _N
_KBN = _K * _BN
_F32 = jnp.float32
_NEG = -3.0e38

_HI = lax.Precision.HIGHEST


def _proj_body(ft, xt, w1a, w1b, w1c, wva, wvb, b1, bv, a_o, cc_o, vl_o, vh_o,
               vc_o):
    f = ft[...]
    x = xt[...]
    xc1 = lax.dot(x, w1c[...], precision=_HI)
    xcv = lax.dot(x, wvb[...], precision=_HI)
    a = lax.dot(f, w1b[...], precision=_HI) + xc1
    a_o[...] = jnp.concatenate([a, jnp.zeros(a.shape, _F32)], axis=1)
    cc_o[...] = lax.dot(f, w1a[...], precision=_HI) - xc1 + b1[...]
    vn = lax.dot(f, wva[...], precision=_HI) + xcv
    vl_o[...] = vn[:, :128]
    vh_o[...] = vn[:, 128:]
    vc_o[...] = bv[...] - xcv


def _projections(ft, xt, w1a, w1b, w1c, wva, wvb, b1, bv):
    rp = 512
    grid = (_BN // rp,)
    full = lambda a: pl.BlockSpec(a.shape, lambda t: (0,) * a.ndim)
    row = lambda d: pl.BlockSpec((rp, d), lambda t: (t, 0))
    return pl.pallas_call(
        _proj_body,
        grid=grid,
        in_specs=[row(_C), row(3), full(w1a), full(w1b), full(w1c),
                  full(wva), full(wvb), full(b1), full(bv)],
        out_specs=[row(2 * _H), row(_H), row(2 * _H), row(2 * _H), row(_C)],
        out_shape=[
            jax.ShapeDtypeStruct((_BN, 2 * _H), _F32),
            jax.ShapeDtypeStruct((_BN, _H), _F32),
            jax.ShapeDtypeStruct((_BN, 2 * _H), _F32),
            jax.ShapeDtypeStruct((_BN, 2 * _H), _F32),
            jax.ShapeDtypeStruct((_BN, _C), _F32),
        ],
    )(ft, xt, w1a, w1b, w1c, wva, wvb, b1, bv)


def _knn_body(xb, xt, idx_o):
    b = pl.program_id(0)
    x_b = xb[0]
    x_f = xt[...]
    g = lax.dot(x_f, x_b, precision=_HI)
    sqc = jnp.sum(x_f * x_f, axis=1, keepdims=True)
    sqr = jnp.sum(x_b * x_b, axis=0, keepdims=True)
    s = -((-2.0 * g + sqr) + sqc)
    iota = lax.broadcasted_iota(jnp.int32, s.shape, 0)
    rows = []
    for _ in range(_K):
        j = jnp.argmax(s, axis=0).astype(jnp.int32)[None, :]
        rows.append(j)
        s = jnp.where(iota == j, _NEG, s)
    idx_o[...] = jnp.concatenate(rows, axis=0) + b * _N


def _knn(xyz, xt):
    rt = 256
    tpb = _N // rt
    return pl.pallas_call(
        _knn_body,
        grid=(_B, tpb),
        in_specs=[
            pl.BlockSpec((1, 3, rt), lambda b, t: (b, 0, t)),
            pl.BlockSpec((_N, 3), lambda b, t: (b, 0)),
        ],
        out_specs=pl.BlockSpec((_K, rt), lambda b, t: (0, b * tpb + t)),
        out_shape=jax.ShapeDtypeStruct((_K, _BN), jnp.int32),
    )(xyz, xt)


def _sc_gather(idx2, a_tab, vl_tab, vh_tab):
    info = plsc.get_sparse_core_info()
    nc, ns = info.num_cores, info.num_subcores
    nw = nc * ns
    ch = 128
    per_w = _KBN // nw
    nch = per_w // ch
    mesh = plsc.VectorSubcoreMesh(core_axis_name="c", subcore_axis_name="s")
    d = 2 * _H
    oshape = jax.ShapeDtypeStruct((_KBN, d), _F32)

    @functools.partial(
        pl.kernel,
        mesh=mesh,
        out_type=(oshape, oshape, oshape),
        scratch_types=[
            pltpu.VMEM((nch, ch), jnp.int32),
            pltpu.VMEM((2, ch, d), _F32),
            pltpu.VMEM((2, ch, d), _F32),
            pltpu.VMEM((2, ch, d), _F32),
            pltpu.SemaphoreType.DMA,
            pltpu.SemaphoreType.DMA,
            pltpu.SemaphoreType.DMA,
            pltpu.SemaphoreType.DMA,
        ],
    )
    def gathk(idx_hbm, a_hbm, vl_hbm, vh_hbm, ag_hbm, vlg_hbm, vhg_hbm,
              idx_all, av, lv, hv, g0, g1, w0, w1):
        wid = lax.axis_index("s") * nc + lax.axis_index("c")
        gsem = (g0, g1)
        wsem = (w0, w1)
        base0 = wid * per_w

        def prime(b, cc):
            ix = idx_all.at[cc]
            pltpu.make_async_copy(a_hbm.at[ix], av.at[b], gsem[b]).start()
            pltpu.make_async_copy(vl_hbm.at[ix], lv.at[b], gsem[b]).start()
            pltpu.make_async_copy(vh_hbm.at[ix], hv.at[b], gsem[b]).start()

        def gwait(b, cc):
            ix = idx_all.at[cc]
            pltpu.make_async_copy(a_hbm.at[ix], av.at[b], gsem[b]).wait()
            pltpu.make_async_copy(vl_hbm.at[ix], lv.at[b], gsem[b]).wait()
            pltpu.make_async_copy(vh_hbm.at[ix], hv.at[b], gsem[b]).wait()

        def wstart(b, cc):
            base = base0 + cc * ch
            pltpu.make_async_copy(av.at[b], ag_hbm.at[pl.ds(base, ch)], wsem[b]).start()
            pltpu.make_async_copy(lv.at[b], vlg_hbm.at[pl.ds(base, ch)], wsem[b]).start()
            pltpu.make_async_copy(hv.at[b], vhg_hbm.at[pl.ds(base, ch)], wsem[b]).start()

        def wwait(b, cc):
            base = base0 + cc * ch
            pltpu.make_async_copy(av.at[b], ag_hbm.at[pl.ds(base, ch)], wsem[b]).wait()
            pltpu.make_async_copy(lv.at[b], vlg_hbm.at[pl.ds(base, ch)], wsem[b]).wait()
            pltpu.make_async_copy(hv.at[b], vhg_hbm.at[pl.ds(base, ch)], wsem[b]).wait()

        pltpu.sync_copy(idx_hbm.at[pl.ds(wid * nch, nch)], idx_all)
        prime(0, 0)
        gwait(0, 0)
        wstart(0, 0)
        prime(1, 1)

        @pl.loop(1, nch - 1, step=2)
        def _pair(c):
            for b2 in range(2):
                cc = c + b2
                buf = (1 + b2) % 2
                oth = 1 - buf
                gwait(buf, cc)
                wstart(buf, cc)
                wwait(oth, cc - 1)
                prime(oth, cc + 1)

        gwait(1, nch - 1)
        wstart(1, nch - 1)
        wwait(0, nch - 2)
        wwait(1, nch - 1)

    return gathk(idx2, a_tab, vl_tab, vh_tab)


def _stats_body(*refs):
    ag = refs[:_K]
    vl = refs[_K:2 * _K]
    vh = refs[2 * _K:3 * _K]
    cc, vc, o1, ov = refs[3 * _K:]

    @pl.when(pl.program_id(0) == 0)
    def _():
        o1[...] = jnp.zeros_like(o1)
        ov[...] = jnp.zeros_like(ov)

    c = cc[...]
    v = vc[...]
    su1 = jnp.zeros((1, _H), _F32)
    sq1 = jnp.zeros((1, _H), _F32)
    suv = jnp.zeros((1, _C), _F32)
    sqv = jnp.zeros((1, _C), _F32)
    for k in range(_K):
        x1 = ag[k][:, :_H] + c
        su1 += jnp.sum(x1, axis=0, keepdims=True)
        sq1 += jnp.sum(x1 * x1, axis=0, keepdims=True)
        v1 = jnp.concatenate([vl[k][...], vh[k][...]], axis=1) + v
        suv += jnp.sum(v1, axis=0, keepdims=True)
        sqv += jnp.sum(v1 * v1, axis=0, keepdims=True)
    z1 = jnp.zeros((6, _H), _F32)
    zv = jnp.zeros((6, _C), _F32)
    o1[...] += jnp.concatenate([su1, sq1, z1], axis=0)
    ov[...] += jnp.concatenate([suv, sqv, zv], axis=0)


def _kspecs(rows, d):
    tpk = _BN // rows
    return [pl.BlockSpec((rows, d), lambda t, kk=k: (kk * tpk + t, 0))
            for k in range(_K)]


def _stats(agf, vlgf, vhgf, cc, vc):
    rs = 512
    grid = (_BN // rs,)
    return pl.pallas_call(
        _stats_body,
        grid=grid,
        in_specs=(_kspecs(rs, 2 * _H) + _kspecs(rs, 2 * _H)
                  + _kspecs(rs, 2 * _H) + [
            pl.BlockSpec((rs, _H), lambda t: (t, 0)),
            pl.BlockSpec((rs, _C), lambda t: (t, 0)),
        ]),
        out_specs=[
            pl.BlockSpec((8, _H), lambda t: (0, 0)),
            pl.BlockSpec((8, _C), lambda t: (0, 0)),
        ],
        out_shape=[
            jax.ShapeDtypeStruct((8, _H), _F32),
            jax.ShapeDtypeStruct((8, _C), _F32),
        ],
    )(*([agf] * _K + [vlgf] * _K + [vhgf] * _K + [cc, vc]))


def _main_body(*refs):
    ag = refs[:_K]
    vl = refs[_K:2 * _K]
    vh = refs[2 * _K:3 * _K]
    cc, vc, sc1, sh1, scv, shv, w2, wo, bo, z_o, zs_o = refs[3 * _K:]
    c = cc[...]
    v = vc[...]
    a1 = sc1[...]
    b1 = sh1[...]
    av = scv[...]
    bv = shv[...]
    w2r = w2[...]
    logits = []
    for k in range(_K):
        h = jnp.maximum((ag[k][:, :_H] + c) * a1 + b1, 0.0)
        logits.append(jnp.sum(h * w2r, axis=1, keepdims=True))
    lg = jnp.concatenate(logits, axis=1)
    mx = jnp.max(lg, axis=1, keepdims=True)
    e = jnp.exp(lg - mx)
    attn = e / jnp.sum(e, axis=1, keepdims=True)
    out = jnp.zeros(v.shape, _F32)
    for k in range(_K):
        v1 = jnp.concatenate([vl[k][...], vh[k][...]], axis=1) + v
        vals = jnp.maximum(v1 * av + bv, 0.0)
        out += attn[:, k:k + 1] * vals
    z = lax.dot(out, wo[...], precision=_HI) + bo[...]
    z_o[...] = z

    @pl.when(pl.program_id(0) == 0)
    def _():
        zs_o[...] = jnp.zeros_like(zs_o)

    zs_o[...] += jnp.concatenate(
        [jnp.sum(z, axis=0, keepdims=True),
         jnp.sum(z * z, axis=0, keepdims=True),
         jnp.zeros((6, _C), _F32)], axis=0)


def _main(agf, vlgf, vhgf, cc, vc, sc1, sh1, scv, shv, w2, wo, bo):
    rm = 256
    grid = (_BN // rm,)
    full = lambda a: pl.BlockSpec(a.shape, lambda t: (0,) * a.ndim)
    return pl.pallas_call(
        _main_body,
        grid=grid,
        in_specs=(_kspecs(rm, 2 * _H) + _kspecs(rm, 2 * _H)
                  + _kspecs(rm, 2 * _H) + [
            pl.BlockSpec((rm, _H), lambda t: (t, 0)),
            pl.BlockSpec((rm, _C), lambda t: (t, 0)),
            full(sc1), full(sh1), full(scv), full(shv), full(w2),
            full(wo), full(bo),
        ]),
        out_specs=[
            pl.BlockSpec((rm, _C), lambda t: (t, 0)),
            pl.BlockSpec((8, _C), lambda t: (0, 0)),
        ],
        out_shape=[
            jax.ShapeDtypeStruct((_BN, _C), _F32),
            jax.ShapeDtypeStruct((8, _C), _F32),
        ],
    )(*([agf] * _K + [vlgf] * _K + [vhgf] * _K +
        [cc, vc, sc1, sh1, scv, shv, w2, wo, bo]))


def _epi_body(z, ft, sco, sho, o_o):
    o_o[...] = jnp.maximum(z[...] * sco[...] + sho[...], 0.0) + ft[...]


def _epilogue(z, ft, sco, sho):
    re = 512
    full = lambda a: pl.BlockSpec(a.shape, lambda t: (0,) * a.ndim)
    return pl.pallas_call(
        _epi_body,
        grid=(_BN // re,),
        in_specs=[
            pl.BlockSpec((re, _C), lambda t: (t, 0)),
            pl.BlockSpec((re, _C), lambda t: (t, 0)),
            full(sco), full(sho),
        ],
        out_specs=pl.BlockSpec((re, _C), lambda t: (t, 0)),
        out_shape=jax.ShapeDtypeStruct((_BN, _C), _F32),
    )(z, ft, sco, sho)


def _bn_coef(s, q, m, g, be):
    mean = s / m
    var = q / m - mean * mean
    sc = g / jnp.sqrt(var + 1e-5)
    return sc, be - mean * sc


def kernel(xyz, feats, W1, b1, g1, be1, W2, b2, Wv, bv, gv, bev, Wo, bo, go, beo):
    ft = jnp.transpose(feats, (0, 2, 1)).reshape(_BN, _C)
    xt = jnp.transpose(xyz, (0, 2, 1)).reshape(_BN, 3)

    w1a = jnp.transpose(W1[:, :_C])
    w1b = jnp.transpose(W1[:, _C:2 * _C])
    w1c = jnp.transpose(W1[:, 2 * _C:])
    wva = jnp.transpose(Wv[:, :_C])
    wvb = jnp.transpose(Wv[:, _C:])

    a_tab, cc_tab, vl_tab, vh_tab, vc_tab = _projections(
        ft, xt, w1a, w1b, w1c, wva, wvb,
        b1.reshape(1, _H), bv.reshape(1, _C))

    idx = _knn(xyz, xt)
    idx2 = idx.reshape(_KBN // 128, 128)

    agf, vlgf, vhgf = _sc_gather(idx2, a_tab, vl_tab, vh_tab)

    o1, ov = _stats(agf, vlgf, vhgf, cc_tab, vc_tab)
    m1 = float(_KBN)
    sc1, sh1 = _bn_coef(o1[0:1], o1[1:2], m1, g1.reshape(1, _H), be1.reshape(1, _H))
    scv, shv = _bn_coef(ov[0:1], ov[1:2], m1, gv.reshape(1, _C), bev.reshape(1, _C))

    z, zs = _main(agf, vlgf, vhgf, cc_tab, vc_tab, sc1, sh1, scv, shv,
                  W2.reshape(1, _H), jnp.transpose(Wo), bo.reshape(1, _C))
    sco, sho = _bn_coef(zs[0:1], zs[1:2], float(_BN),
                        go.reshape(1, _C), beo.reshape(1, _C))

    o = _epilogue(z, ft, sco, sho)
    return jnp.transpose(o.reshape(_B, _N, _C), (0, 2, 1))


# speedup vs baseline: 11.8932x; 1.1119x over previous
import functools

import jax
import jax.numpy as jnp
from jax import lax
from jax.experimental import pallas as pl
from jax.experimental.pallas import tpu as pltpu
from jax.experimental.pallas import tpu_sc as plsc

_B, _N, _C, _H, _K = 8, 1024, 256, 64, 16
_BN = _B * _N
_KBN = _K * _BN
_HBN = _BN // 2
_HKBN = _K * _HBN
_F32 = jnp.float32
_NEG = -3.0e38

_HI = lax.Precision.HIGHEST


def _proj_body(ft, xt, w1a, w1b, w1c, wva, wvb, b1, bv, a_o, cc_o, vl_o, vh_o,
               vc_o):
    f = ft[...]
    x = xt[...]
    xc1 = lax.dot(x, w1c[...], precision=_HI)
    xcv = lax.dot(x, wvb[...], precision=_HI)
    a = lax.dot(f, w1b[...], precision=_HI) + xc1
    a_o[...] = jnp.concatenate([a, jnp.zeros(a.shape, _F32)], axis=1)
    cc_o[...] = lax.dot(f, w1a[...], precision=_HI) - xc1 + b1[...]
    vn = lax.dot(f, wva[...], precision=_HI) + xcv
    vl_o[...] = vn[:, :128]
    vh_o[...] = vn[:, 128:]
    vc_o[...] = bv[...] - xcv


def _projections(ft, xt, w1a, w1b, w1c, wva, wvb, b1, bv):
    rp = 512
    grid = (_BN // rp,)
    full = lambda a: pl.BlockSpec(a.shape, lambda t: (0,) * a.ndim)
    row = lambda d: pl.BlockSpec((rp, d), lambda t: (t, 0))
    return pl.pallas_call(
        _proj_body,
        grid=grid,
        in_specs=[row(_C), row(3), full(w1a), full(w1b), full(w1c),
                  full(wva), full(wvb), full(b1), full(bv)],
        out_specs=[row(2 * _H), row(_H), row(2 * _H), row(2 * _H), row(_C)],
        out_shape=[
            jax.ShapeDtypeStruct((_BN, 2 * _H), _F32),
            jax.ShapeDtypeStruct((_BN, _H), _F32),
            jax.ShapeDtypeStruct((_BN, 2 * _H), _F32),
            jax.ShapeDtypeStruct((_BN, 2 * _H), _F32),
            jax.ShapeDtypeStruct((_BN, _C), _F32),
        ],
    )(ft, xt, w1a, w1b, w1c, wva, wvb, b1, bv)


def _knn_body(xb, xt, idx_o, b0):
    b = b0 + pl.program_id(0)
    x_b = xb[0]
    x_f = xt[...]
    g = lax.dot(x_f, x_b, precision=_HI)
    sqc = jnp.sum(x_f * x_f, axis=1, keepdims=True)
    sqr = jnp.sum(x_b * x_b, axis=0, keepdims=True)
    s = -((-2.0 * g + sqr) + sqc)
    iota = lax.broadcasted_iota(jnp.int32, s.shape, 0)
    rows = []
    for _ in range(_K):
        j = jnp.argmax(s, axis=0).astype(jnp.int32)[None, :]
        rows.append(j)
        s = jnp.where(iota == j, _NEG, s)
    idx_o[...] = jnp.concatenate(rows, axis=0) + b * _N


def _knn(xyz, xt, h):
    rt = 256
    tpb = _N // rt
    hb = _B // 2
    b0 = h * hb

    def body(xb, xtr, idx_o):
        _knn_body(xb, xtr, idx_o, b0)

    return pl.pallas_call(
        body,
        grid=(hb, tpb),
        in_specs=[
            pl.BlockSpec((1, 3, rt), lambda b, t: (b0 + b, 0, t)),
            pl.BlockSpec((_N, 3), lambda b, t: (b0 + b, 0)),
        ],
        out_specs=pl.BlockSpec((_K, rt), lambda b, t: (0, b * tpb + t)),
        out_shape=jax.ShapeDtypeStruct((_K, _HBN), jnp.int32),
    )(xyz, xt)


def _sc_gather(idx2, a_tab, vl_tab, vh_tab):
    info = plsc.get_sparse_core_info()
    nc, ns = info.num_cores, info.num_subcores
    nw = nc * ns
    ch = 128
    nrow = idx2.shape[0] * ch
    per_w = nrow // nw
    nch = per_w // ch
    mesh = plsc.VectorSubcoreMesh(core_axis_name="c", subcore_axis_name="s")
    d = 2 * _H
    oshape = jax.ShapeDtypeStruct((nrow, d), _F32)

    @functools.partial(
        pl.kernel,
        mesh=mesh,
        out_type=(oshape, oshape, oshape),
        scratch_types=[
            pltpu.VMEM((nch, ch), jnp.int32),
            pltpu.VMEM((2, ch, d), _F32),
            pltpu.VMEM((2, ch, d), _F32),
            pltpu.VMEM((2, ch, d), _F32),
            pltpu.SemaphoreType.DMA,
            pltpu.SemaphoreType.DMA,
            pltpu.SemaphoreType.DMA,
            pltpu.SemaphoreType.DMA,
        ],
    )
    def gathk(idx_hbm, a_hbm, vl_hbm, vh_hbm, ag_hbm, vlg_hbm, vhg_hbm,
              idx_all, av, lv, hv, g0, g1, w0, w1):
        wid = lax.axis_index("s") * nc + lax.axis_index("c")
        gsem = (g0, g1)
        wsem = (w0, w1)
        base0 = wid * per_w

        def prime(b, cc):
            ix = idx_all.at[cc]
            pltpu.make_async_copy(a_hbm.at[ix], av.at[b], gsem[b]).start()
            pltpu.make_async_copy(vl_hbm.at[ix], lv.at[b], gsem[b]).start()
            pltpu.make_async_copy(vh_hbm.at[ix], hv.at[b], gsem[b]).start()

        def gwait(b, cc):
            ix = idx_all.at[cc]
            pltpu.make_async_copy(a_hbm.at[ix], av.at[b], gsem[b]).wait()
            pltpu.make_async_copy(vl_hbm.at[ix], lv.at[b], gsem[b]).wait()
            pltpu.make_async_copy(vh_hbm.at[ix], hv.at[b], gsem[b]).wait()

        def wstart(b, cc):
            base = base0 + cc * ch
            pltpu.make_async_copy(av.at[b], ag_hbm.at[pl.ds(base, ch)], wsem[b]).start()
            pltpu.make_async_copy(lv.at[b], vlg_hbm.at[pl.ds(base, ch)], wsem[b]).start()
            pltpu.make_async_copy(hv.at[b], vhg_hbm.at[pl.ds(base, ch)], wsem[b]).start()

        def wwait(b, cc):
            base = base0 + cc * ch
            pltpu.make_async_copy(av.at[b], ag_hbm.at[pl.ds(base, ch)], wsem[b]).wait()
            pltpu.make_async_copy(lv.at[b], vlg_hbm.at[pl.ds(base, ch)], wsem[b]).wait()
            pltpu.make_async_copy(hv.at[b], vhg_hbm.at[pl.ds(base, ch)], wsem[b]).wait()

        pltpu.sync_copy(idx_hbm.at[pl.ds(wid * nch, nch)], idx_all)
        prime(0, 0)
        gwait(0, 0)
        wstart(0, 0)
        prime(1, 1)

        @pl.loop(1, nch - 1, step=2)
        def _pair(c):
            for b2 in range(2):
                cc = c + b2
                buf = (1 + b2) % 2
                oth = 1 - buf
                gwait(buf, cc)
                wstart(buf, cc)
                wwait(oth, cc - 1)
                prime(oth, cc + 1)

        gwait(1, nch - 1)
        wstart(1, nch - 1)
        wwait(0, nch - 2)
        wwait(1, nch - 1)

    return gathk(idx2, a_tab, vl_tab, vh_tab)


def _stats_body(*refs):
    ag = refs[:_K]
    vl = refs[_K:2 * _K]
    vh = refs[2 * _K:3 * _K]
    cc, vc, o1, ov = refs[3 * _K:]

    @pl.when(pl.program_id(0) == 0)
    def _():
        o1[...] = jnp.zeros_like(o1)
        ov[...] = jnp.zeros_like(ov)

    c = cc[...]
    v = vc[...]
    su1 = jnp.zeros((1, _H), _F32)
    sq1 = jnp.zeros((1, _H), _F32)
    suv = jnp.zeros((1, _C), _F32)
    sqv = jnp.zeros((1, _C), _F32)
    for k in range(_K):
        x1 = ag[k][:, :_H] + c
        su1 += jnp.sum(x1, axis=0, keepdims=True)
        sq1 += jnp.sum(x1 * x1, axis=0, keepdims=True)
        v1 = jnp.concatenate([vl[k][...], vh[k][...]], axis=1) + v
        suv += jnp.sum(v1, axis=0, keepdims=True)
        sqv += jnp.sum(v1 * v1, axis=0, keepdims=True)
    z1 = jnp.zeros((6, _H), _F32)
    zv = jnp.zeros((6, _C), _F32)
    o1[...] += jnp.concatenate([su1, sq1, z1], axis=0)
    ov[...] += jnp.concatenate([suv, sqv, zv], axis=0)


def _kspecs(rows, d):
    tpk = _HBN // rows
    return [pl.BlockSpec((rows, d), lambda t, kk=k: (kk * tpk + t, 0))
            for k in range(_K)]


def _stats(agf, vlgf, vhgf, cc, vc, h):
    rs = 512
    grid = (_HBN // rs,)
    off = h * (_HBN // rs)
    return pl.pallas_call(
        _stats_body,
        grid=grid,
        in_specs=(_kspecs(rs, 2 * _H) + _kspecs(rs, 2 * _H)
                  + _kspecs(rs, 2 * _H) + [
            pl.BlockSpec((rs, _H), lambda t: (off + t, 0)),
            pl.BlockSpec((rs, _C), lambda t: (off + t, 0)),
        ]),
        out_specs=[
            pl.BlockSpec((8, _H), lambda t: (0, 0)),
            pl.BlockSpec((8, _C), lambda t: (0, 0)),
        ],
        out_shape=[
            jax.ShapeDtypeStruct((8, _H), _F32),
            jax.ShapeDtypeStruct((8, _C), _F32),
        ],
    )(*([agf] * _K + [vlgf] * _K + [vhgf] * _K + [cc, vc]))


def _main_body(*refs):
    ag = refs[:_K]
    vl = refs[_K:2 * _K]
    vh = refs[2 * _K:3 * _K]
    cc, vc, sc1, sh1, scv, shv, w2, wo, bo, z_o, zs_o = refs[3 * _K:]
    c = cc[...]
    v = vc[...]
    a1 = sc1[...]
    b1 = sh1[...]
    av = scv[...]
    bv = shv[...]
    w2r = w2[...]
    logits = []
    for k in range(_K):
        h = jnp.maximum((ag[k][:, :_H] + c) * a1 + b1, 0.0)
        logits.append(jnp.sum(h * w2r, axis=1, keepdims=True))
    lg = jnp.concatenate(logits, axis=1)
    mx = jnp.max(lg, axis=1, keepdims=True)
    e = jnp.exp(lg - mx)
    attn = e / jnp.sum(e, axis=1, keepdims=True)
    out = jnp.zeros(v.shape, _F32)
    for k in range(_K):
        v1 = jnp.concatenate([vl[k][...], vh[k][...]], axis=1) + v
        vals = jnp.maximum(v1 * av + bv, 0.0)
        out += attn[:, k:k + 1] * vals
    z = lax.dot(out, wo[...], precision=_HI) + bo[...]
    z_o[...] = z

    @pl.when(pl.program_id(0) == 0)
    def _():
        zs_o[...] = jnp.zeros_like(zs_o)

    zs_o[...] += jnp.concatenate(
        [jnp.sum(z, axis=0, keepdims=True),
         jnp.sum(z * z, axis=0, keepdims=True),
         jnp.zeros((6, _C), _F32)], axis=0)


def _main(agf, vlgf, vhgf, cc, vc, sc1, sh1, scv, shv, w2, wo, bo, h):
    rm = 256
    grid = (_HBN // rm,)
    off = h * (_HBN // rm)
    full = lambda a: pl.BlockSpec(a.shape, lambda t: (0,) * a.ndim)
    return pl.pallas_call(
        _main_body,
        grid=grid,
        in_specs=(_kspecs(rm, 2 * _H) + _kspecs(rm, 2 * _H)
                  + _kspecs(rm, 2 * _H) + [
            pl.BlockSpec((rm, _H), lambda t: (off + t, 0)),
            pl.BlockSpec((rm, _C), lambda t: (off + t, 0)),
            full(sc1), full(sh1), full(scv), full(shv), full(w2),
            full(wo), full(bo),
        ]),
        out_specs=[
            pl.BlockSpec((rm, _C), lambda t: (t, 0)),
            pl.BlockSpec((8, _C), lambda t: (0, 0)),
        ],
        out_shape=[
            jax.ShapeDtypeStruct((_HBN, _C), _F32),
            jax.ShapeDtypeStruct((8, _C), _F32),
        ],
    )(*([agf] * _K + [vlgf] * _K + [vhgf] * _K +
        [cc, vc, sc1, sh1, scv, shv, w2, wo, bo]))


def _epi_body(z, ft, sco, sho, o_o):
    o_o[...] = jnp.maximum(z[...] * sco[...] + sho[...], 0.0) + ft[...]


def _epilogue(z, ft, sco, sho):
    re = 512
    full = lambda a: pl.BlockSpec(a.shape, lambda t: (0,) * a.ndim)
    return pl.pallas_call(
        _epi_body,
        grid=(_BN // re,),
        in_specs=[
            pl.BlockSpec((re, _C), lambda t: (t, 0)),
            pl.BlockSpec((re, _C), lambda t: (t, 0)),
            full(sco), full(sho),
        ],
        out_specs=pl.BlockSpec((re, _C), lambda t: (t, 0)),
        out_shape=jax.ShapeDtypeStruct((_BN, _C), _F32),
    )(z, ft, sco, sho)


def _bn_coef(s, q, m, g, be):
    mean = s / m
    var = q / m - mean * mean
    sc = g / jnp.sqrt(var + 1e-5)
    return sc, be - mean * sc


def kernel(xyz, feats, W1, b1, g1, be1, W2, b2, Wv, bv, gv, bev, Wo, bo, go, beo):
    ft = jnp.transpose(feats, (0, 2, 1)).reshape(_BN, _C)
    xt = jnp.transpose(xyz, (0, 2, 1)).reshape(_BN, 3)

    w1a = jnp.transpose(W1[:, :_C])
    w1b = jnp.transpose(W1[:, _C:2 * _C])
    w1c = jnp.transpose(W1[:, 2 * _C:])
    wva = jnp.transpose(Wv[:, :_C])
    wvb = jnp.transpose(Wv[:, _C:])

    a_tab, cc_tab, vl_tab, vh_tab, vc_tab = _projections(
        ft, xt, w1a, w1b, w1c, wva, wvb,
        b1.reshape(1, _H), bv.reshape(1, _C))

    idx0 = _knn(xyz, xt, 0)
    ga = _sc_gather(idx0.reshape(_HKBN // 128, 128), a_tab, vl_tab, vh_tab)
    idx1 = _knn(xyz, xt, 1)
    gb = _sc_gather(idx1.reshape(_HKBN // 128, 128), a_tab, vl_tab, vh_tab)

    s0 = _stats(*ga, cc_tab, vc_tab, 0)
    s1 = _stats(*gb, cc_tab, vc_tab, 1)
    o1 = s0[0] + s1[0]
    ov = s0[1] + s1[1]
    m1 = float(_KBN)
    sc1, sh1 = _bn_coef(o1[0:1], o1[1:2], m1, g1.reshape(1, _H), be1.reshape(1, _H))
    scv, shv = _bn_coef(ov[0:1], ov[1:2], m1, gv.reshape(1, _C), bev.reshape(1, _C))

    w2r = W2.reshape(1, _H)
    wot = jnp.transpose(Wo)
    bor = bo.reshape(1, _C)
    z0, zs0 = _main(*ga, cc_tab, vc_tab, sc1, sh1, scv, shv, w2r, wot, bor, 0)
    z1, zs1 = _main(*gb, cc_tab, vc_tab, sc1, sh1, scv, shv, w2r, wot, bor, 1)
    z = jnp.concatenate([z0, z1], axis=0)
    zs = zs0 + zs1
    sco, sho = _bn_coef(zs[0:1], zs[1:2], float(_BN),
                        go.reshape(1, _C), beo.reshape(1, _C))

    o = _epilogue(z, ft, sco, sho)
    return jnp.transpose(o.reshape(_B, _N, _C), (0, 2, 1))


# speedup vs baseline: 13.5986x; 1.1434x over previous
import functools

import jax
import jax.numpy as jnp
from jax import lax
from jax.experimental import pallas as pl
from jax.experimental.pallas import tpu as pltpu
from jax.experimental.pallas import tpu_sc as plsc

_B, _N, _C, _H, _K = 8, 1024, 256, 64, 16
_BN = _B * _N
_KBN = _K * _BN
_HBN = _BN // 2
_HKBN = _K * _HBN
_F32 = jnp.float32
_NEG = -3.0e38

_HI = lax.Precision.HIGHEST


def _proj_body(ft, xt, w1a, w1b, w1c, wva, wvb, b1, bv, a_o, cc_o, vp_o,
               vc_o):
    f = ft[...]
    x = xt[...]
    xc1 = lax.dot(x, w1c[...], precision=_HI)
    xcv = lax.dot(x, wvb[...], precision=_HI)
    a = lax.dot(f, w1b[...], precision=_HI) + xc1
    a_o[...] = jnp.concatenate([a, jnp.zeros(a.shape, _F32)], axis=1)
    cc_o[...] = lax.dot(f, w1a[...], precision=_HI) - xc1 + b1[...]
    vn = lax.dot(f, wva[...], precision=_HI) + xcv
    lo = lax.bitcast_convert_type(
        vn[:, :128].astype(jnp.bfloat16), jnp.uint16).astype(jnp.int32)
    hi = lax.bitcast_convert_type(
        vn[:, 128:].astype(jnp.bfloat16), jnp.uint16).astype(jnp.int32)
    vp_o[...] = jnp.left_shift(hi, 16) | lo
    vc_o[...] = bv[...] - xcv


def _projections(ft, xt, w1a, w1b, w1c, wva, wvb, b1, bv):
    rp = 512
    grid = (_BN // rp,)
    full = lambda a: pl.BlockSpec(a.shape, lambda t: (0,) * a.ndim)
    row = lambda d: pl.BlockSpec((rp, d), lambda t: (t, 0))
    return pl.pallas_call(
        _proj_body,
        grid=grid,
        in_specs=[row(_C), row(3), full(w1a), full(w1b), full(w1c),
                  full(wva), full(wvb), full(b1), full(bv)],
        out_specs=[row(2 * _H), row(_H), row(2 * _H), row(_C)],
        out_shape=[
            jax.ShapeDtypeStruct((_BN, 2 * _H), _F32),
            jax.ShapeDtypeStruct((_BN, _H), _F32),
            jax.ShapeDtypeStruct((_BN, 2 * _H), jnp.int32),
            jax.ShapeDtypeStruct((_BN, _C), _F32),
        ],
    )(ft, xt, w1a, w1b, w1c, wva, wvb, b1, bv)


def _knn_body(xb, xt, idx_o, b0):
    b = b0 + pl.program_id(0)
    x_b = xb[0]
    x_f = xt[...]
    g = lax.dot(x_f, x_b, precision=_HI)
    sqc = jnp.sum(x_f * x_f, axis=1, keepdims=True)
    sqr = jnp.sum(x_b * x_b, axis=0, keepdims=True)
    s = -((-2.0 * g + sqr) + sqc)
    iota = lax.broadcasted_iota(jnp.int32, s.shape, 0)
    rows = []
    for _ in range(_K):
        j = jnp.argmax(s, axis=0).astype(jnp.int32)[None, :]
        rows.append(j)
        s = jnp.where(iota == j, _NEG, s)
    idx_o[...] = jnp.concatenate(rows, axis=0) + b * _N


def _knn(xyz, xt, h):
    rt = 256
    tpb = _N // rt
    hb = _B // 2
    b0 = h * hb

    def body(xb, xtr, idx_o):
        _knn_body(xb, xtr, idx_o, b0)

    return pl.pallas_call(
        body,
        grid=(hb, tpb),
        in_specs=[
            pl.BlockSpec((1, 3, rt), lambda b, t: (b0 + b, 0, t)),
            pl.BlockSpec((_N, 3), lambda b, t: (b0 + b, 0)),
        ],
        out_specs=pl.BlockSpec((_K, rt), lambda b, t: (0, b * tpb + t)),
        out_shape=jax.ShapeDtypeStruct((_K, _HBN), jnp.int32),
    )(xyz, xt)


def _sc_gather(idx2, a_tab, vp_tab):
    info = plsc.get_sparse_core_info()
    nc, ns = info.num_cores, info.num_subcores
    nw = nc * ns
    ch = 128
    nrow = idx2.shape[0] * ch
    per_w = nrow // nw
    nch = per_w // ch
    mesh = plsc.VectorSubcoreMesh(core_axis_name="c", subcore_axis_name="s")
    d = 2 * _H

    @functools.partial(
        pl.kernel,
        mesh=mesh,
        out_type=(
            jax.ShapeDtypeStruct((nrow, d), _F32),
            jax.ShapeDtypeStruct((nrow, d), jnp.int32),
        ),
        scratch_types=[
            pltpu.VMEM((nch, ch), jnp.int32),
            pltpu.VMEM((2, ch, d), _F32),
            pltpu.VMEM((2, ch, d), jnp.int32),
            pltpu.SemaphoreType.DMA,
            pltpu.SemaphoreType.DMA,
            pltpu.SemaphoreType.DMA,
            pltpu.SemaphoreType.DMA,
        ],
    )
    def gathk(idx_hbm, a_hbm, vp_hbm, ag_hbm, vpg_hbm,
              idx_all, av, pv, g0, g1, w0, w1):
        wid = lax.axis_index("s") * nc + lax.axis_index("c")
        gsem = (g0, g1)
        wsem = (w0, w1)
        base0 = wid * per_w

        def prime(b, cc):
            ix = idx_all.at[cc]
            pltpu.make_async_copy(a_hbm.at[ix], av.at[b], gsem[b]).start()
            pltpu.make_async_copy(vp_hbm.at[ix], pv.at[b], gsem[b]).start()

        def gwait(b, cc):
            ix = idx_all.at[cc]
            pltpu.make_async_copy(a_hbm.at[ix], av.at[b], gsem[b]).wait()
            pltpu.make_async_copy(vp_hbm.at[ix], pv.at[b], gsem[b]).wait()

        def wstart(b, cc):
            base = base0 + cc * ch
            pltpu.make_async_copy(av.at[b], ag_hbm.at[pl.ds(base, ch)], wsem[b]).start()
            pltpu.make_async_copy(pv.at[b], vpg_hbm.at[pl.ds(base, ch)], wsem[b]).start()

        def wwait(b, cc):
            base = base0 + cc * ch
            pltpu.make_async_copy(av.at[b], ag_hbm.at[pl.ds(base, ch)], wsem[b]).wait()
            pltpu.make_async_copy(pv.at[b], vpg_hbm.at[pl.ds(base, ch)], wsem[b]).wait()

        pltpu.sync_copy(idx_hbm.at[pl.ds(wid * nch, nch)], idx_all)
        prime(0, 0)
        gwait(0, 0)
        wstart(0, 0)
        prime(1, 1)

        @pl.loop(1, nch - 1, step=2)
        def _pair(c):
            for b2 in range(2):
                cc = c + b2
                buf = (1 + b2) % 2
                oth = 1 - buf
                gwait(buf, cc)
                wstart(buf, cc)
                wwait(oth, cc - 1)
                prime(oth, cc + 1)

        gwait(1, nch - 1)
        wstart(1, nch - 1)
        wwait(0, nch - 2)
        wwait(1, nch - 1)

    return gathk(idx2, a_tab, vp_tab)


def _unpack_v(w):
    vlo = lax.bitcast_convert_type(jnp.left_shift(w, 16), _F32)
    vhi = lax.bitcast_convert_type(jnp.bitwise_and(w, jnp.int32(-65536)), _F32)
    return jnp.concatenate([vlo, vhi], axis=1)


def _stats_body(*refs):
    ag = refs[:_K]
    vp = refs[_K:2 * _K]
    cc, vc, o1, ov = refs[2 * _K:]

    @pl.when(pl.program_id(0) == 0)
    def _():
        o1[...] = jnp.zeros_like(o1)
        ov[...] = jnp.zeros_like(ov)

    c = cc[...]
    v = vc[...]
    su1 = jnp.zeros((1, _H), _F32)
    sq1 = jnp.zeros((1, _H), _F32)
    suv = jnp.zeros((1, _C), _F32)
    sqv = jnp.zeros((1, _C), _F32)
    for k in range(_K):
        x1 = ag[k][:, :_H] + c
        su1 += jnp.sum(x1, axis=0, keepdims=True)
        sq1 += jnp.sum(x1 * x1, axis=0, keepdims=True)
        v1 = _unpack_v(vp[k][...]) + v
        suv += jnp.sum(v1, axis=0, keepdims=True)
        sqv += jnp.sum(v1 * v1, axis=0, keepdims=True)
    z1 = jnp.zeros((6, _H), _F32)
    zv = jnp.zeros((6, _C), _F32)
    o1[...] += jnp.concatenate([su1, sq1, z1], axis=0)
    ov[...] += jnp.concatenate([suv, sqv, zv], axis=0)


def _kspecs(rows, d):
    tpk = _HBN // rows
    return [pl.BlockSpec((rows, d), lambda t, kk=k: (kk * tpk + t, 0))
            for k in range(_K)]


def _stats(agf, vpgf, cc, vc, h):
    rs = 512
    grid = (_HBN // rs,)
    off = h * (_HBN // rs)
    return pl.pallas_call(
        _stats_body,
        grid=grid,
        in_specs=(_kspecs(rs, 2 * _H) + _kspecs(rs, 2 * _H) + [
            pl.BlockSpec((rs, _H), lambda t: (off + t, 0)),
            pl.BlockSpec((rs, _C), lambda t: (off + t, 0)),
        ]),
        out_specs=[
            pl.BlockSpec((8, _H), lambda t: (0, 0)),
            pl.BlockSpec((8, _C), lambda t: (0, 0)),
        ],
        out_shape=[
            jax.ShapeDtypeStruct((8, _H), _F32),
            jax.ShapeDtypeStruct((8, _C), _F32),
        ],
    )(*([agf] * _K + [vpgf] * _K + [cc, vc]))


def _main_body(*refs):
    ag = refs[:_K]
    vp = refs[_K:2 * _K]
    cc, vc, sc1, sh1, scv, shv, w2, wo, bo, z_o, zs_o = refs[2 * _K:]
    c = cc[...]
    v = vc[...]
    a1 = sc1[...]
    b1 = sh1[...]
    av = scv[...]
    bv = shv[...]
    w2r = w2[...]
    logits = []
    for k in range(_K):
        h = jnp.maximum((ag[k][:, :_H] + c) * a1 + b1, 0.0)
        logits.append(jnp.sum(h * w2r, axis=1, keepdims=True))
    lg = jnp.concatenate(logits, axis=1)
    mx = jnp.max(lg, axis=1, keepdims=True)
    e = jnp.exp(lg - mx)
    attn = e / jnp.sum(e, axis=1, keepdims=True)
    out = jnp.zeros(v.shape, _F32)
    for k in range(_K):
        v1 = _unpack_v(vp[k][...]) + v
        vals = jnp.maximum(v1 * av + bv, 0.0)
        out += attn[:, k:k + 1] * vals
    z = lax.dot(out, wo[...], precision=_HI) + bo[...]
    z_o[...] = z

    @pl.when(pl.program_id(0) == 0)
    def _():
        zs_o[...] = jnp.zeros_like(zs_o)

    zs_o[...] += jnp.concatenate(
        [jnp.sum(z, axis=0, keepdims=True),
         jnp.sum(z * z, axis=0, keepdims=True),
         jnp.zeros((6, _C), _F32)], axis=0)


def _main(agf, vpgf, cc, vc, sc1, sh1, scv, shv, w2, wo, bo, h):
    rm = 256
    grid = (_HBN // rm,)
    off = h * (_HBN // rm)
    full = lambda a: pl.BlockSpec(a.shape, lambda t: (0,) * a.ndim)
    return pl.pallas_call(
        _main_body,
        grid=grid,
        in_specs=(_kspecs(rm, 2 * _H) + _kspecs(rm, 2 * _H) + [
            pl.BlockSpec((rm, _H), lambda t: (off + t, 0)),
            pl.BlockSpec((rm, _C), lambda t: (off + t, 0)),
            full(sc1), full(sh1), full(scv), full(shv), full(w2),
            full(wo), full(bo),
        ]),
        out_specs=[
            pl.BlockSpec((rm, _C), lambda t: (t, 0)),
            pl.BlockSpec((8, _C), lambda t: (0, 0)),
        ],
        out_shape=[
            jax.ShapeDtypeStruct((_HBN, _C), _F32),
            jax.ShapeDtypeStruct((8, _C), _F32),
        ],
    )(*([agf] * _K + [vpgf] * _K +
        [cc, vc, sc1, sh1, scv, shv, w2, wo, bo]))


def _epi_body(z, ft, sco, sho, o_o):
    o_o[...] = jnp.maximum(z[...] * sco[...] + sho[...], 0.0) + ft[...]


def _epilogue(z, ft, sco, sho):
    re = 512
    full = lambda a: pl.BlockSpec(a.shape, lambda t: (0,) * a.ndim)
    return pl.pallas_call(
        _epi_body,
        grid=(_BN // re,),
        in_specs=[
            pl.BlockSpec((re, _C), lambda t: (t, 0)),
            pl.BlockSpec((re, _C), lambda t: (t, 0)),
            full(sco), full(sho),
        ],
        out_specs=pl.BlockSpec((re, _C), lambda t: (t, 0)),
        out_shape=jax.ShapeDtypeStruct((_BN, _C), _F32),
    )(z, ft, sco, sho)


def _bn_coef(s, q, m, g, be):
    mean = s / m
    var = q / m - mean * mean
    sc = g / jnp.sqrt(var + 1e-5)
    return sc, be - mean * sc


def kernel(xyz, feats, W1, b1, g1, be1, W2, b2, Wv, bv, gv, bev, Wo, bo, go, beo):
    ft = jnp.transpose(feats, (0, 2, 1)).reshape(_BN, _C)
    xt = jnp.transpose(xyz, (0, 2, 1)).reshape(_BN, 3)

    w1a = jnp.transpose(W1[:, :_C])
    w1b = jnp.transpose(W1[:, _C:2 * _C])
    w1c = jnp.transpose(W1[:, 2 * _C:])
    wva = jnp.transpose(Wv[:, :_C])
    wvb = jnp.transpose(Wv[:, _C:])

    a_tab, cc_tab, vp_tab, vc_tab = _projections(
        ft, xt, w1a, w1b, w1c, wva, wvb,
        b1.reshape(1, _H), bv.reshape(1, _C))

    idx0 = _knn(xyz, xt, 0)
    ga = _sc_gather(idx0.reshape(_HKBN // 128, 128), a_tab, vp_tab)
    idx1 = _knn(xyz, xt, 1)
    gb = _sc_gather(idx1.reshape(_HKBN // 128, 128), a_tab, vp_tab)

    s0 = _stats(*ga, cc_tab, vc_tab, 0)
    s1 = _stats(*gb, cc_tab, vc_tab, 1)
    o1 = s0[0] + s1[0]
    ov = s0[1] + s1[1]
    m1 = float(_KBN)
    sc1, sh1 = _bn_coef(o1[0:1], o1[1:2], m1, g1.reshape(1, _H), be1.reshape(1, _H))
    scv, shv = _bn_coef(ov[0:1], ov[1:2], m1, gv.reshape(1, _C), bev.reshape(1, _C))

    w2r = W2.reshape(1, _H)
    wot = jnp.transpose(Wo)
    bor = bo.reshape(1, _C)
    z0, zs0 = _main(*ga, cc_tab, vc_tab, sc1, sh1, scv, shv, w2r, wot, bor, 0)
    z1, zs1 = _main(*gb, cc_tab, vc_tab, sc1, sh1, scv, shv, w2r, wot, bor, 1)
    z = jnp.concatenate([z0, z1], axis=0)
    zs = zs0 + zs1
    sco, sho = _bn_coef(zs[0:1], zs[1:2], float(_BN),
                        go.reshape(1, _C), beo.reshape(1, _C))

    o = _epilogue(z, ft, sco, sho)
    return jnp.transpose(o.reshape(_B, _N, _C), (0, 2, 1))
